# baseline (speedup 1.0000x reference)
import jax
import jax.numpy as jnp
from jax import lax
from jax.experimental import pallas as pl
from jax.experimental.pallas import tpu as pltpu
from jax.experimental.pallas import tpu_sc as plsc

BATCH = 4
SEQ = 2048
D = 768
TOKENS = BATCH * SEQ
NUM_WORKERS = 32
S_PER_W = SEQ // NUM_WORKERS
CHUNK = 16
CH_PER_B = S_PER_W // CHUNK
NUM_CHUNKS = BATCH * CH_PER_B
GBUF = 3
OBUF = 3
AHEAD = 2

_mesh = plsc.VectorSubcoreMesh(core_axis_name="c", subcore_axis_name="s")

_scratch = (
    [pltpu.VMEM((S_PER_W * BATCH,), jnp.int32)]
    + [pltpu.VMEM((S_PER_W, D), jnp.float32)]
    + [pltpu.VMEM((CHUNK, D), jnp.float32) for _ in range(GBUF)]
    + [pltpu.VMEM((CHUNK, D), jnp.float32) for _ in range(OBUF)]
    + [pltpu.SemaphoreType.DMA for _ in range(GBUF)]
    + [pltpu.SemaphoreType.DMA for _ in range(OBUF)]
    + [pltpu.SemaphoreType.DMA]
    + [pltpu.SemaphoreType.DMA]
)


@pl.kernel(
    out_type=jax.ShapeDtypeStruct((TOKENS, D), jnp.float32),
    mesh=_mesh,
    scratch_types=_scratch,
)
def _embed(ids_hbm, table_hbm, pos_hbm, out_hbm, idx_v, pbuf,
           g0, g1, g2, o0, o1, o2,
           gs0, gs1, gs2, ws0, ws1, ws2, isem, psem):
    gbufs = (g0, g1, g2)
    obufs = (o0, o1, o2)
    gsem = (gs0, gs1, gs2)
    wsem = (ws0, ws1, ws2)
    wid = lax.axis_index("s") * 2 + lax.axis_index("c")
    s_base = wid * S_PER_W

    id_copies = [
        pltpu.async_copy(
            ids_hbm.at[pl.ds(b * SEQ + s_base, S_PER_W)],
            idx_v.at[pl.ds(b * S_PER_W, S_PER_W)], isem)
        for b in range(BATCH)
    ]
    pos_copy = pltpu.async_copy(pos_hbm.at[pl.ds(s_base, S_PER_W)], pbuf, psem)
    for c in id_copies:
        c.wait()

    def start_gather(j):
        slot = j % GBUF
        return pltpu.async_copy(
            table_hbm.at[idx_v.at[pl.ds(j * CHUNK, CHUNK)]],
            gbufs[slot], gsem[slot],
        )

    def out_base(j):
        b, c = divmod(j, CH_PER_B)
        return b * SEQ + s_base + c * CHUNK

    gathers = [None] * NUM_CHUNKS
    writes = [None] * NUM_CHUNKS
    for j in range(min(AHEAD, NUM_CHUNKS)):
        gathers[j] = start_gather(j)
    pos_copy.wait()

    for j in range(NUM_CHUNKS):
        gathers[j].wait()
        if j - OBUF >= 0:
            writes[j - OBUF].wait()
        gbuf = gbufs[j % GBUF]
        obuf = obufs[j % OBUF]
        prow = (j % CH_PER_B) * CHUNK

        def add_row(r, _):
            for k in range(D // 16):
                sl = pl.ds(k * 16, 16)
                obuf[r, sl] = gbuf[r, sl] + pbuf[prow + r, sl]
            return 0

        lax.fori_loop(0, CHUNK, add_row, 0, unroll=2)
        writes[j] = pltpu.async_copy(
            obuf, out_hbm.at[pl.ds(out_base(j), CHUNK)], wsem[j % OBUF])
        nxt = j + AHEAD
        if nxt < NUM_CHUNKS:
            gathers[nxt] = start_gather(nxt)

    for j in range(max(0, NUM_CHUNKS - OBUF), NUM_CHUNKS):
        writes[j].wait()


def kernel(token_ids, token_table, pos_table):
    out = _embed(token_ids.reshape(TOKENS), token_table, pos_table)
    return out.reshape(BATCH, SEQ, D)


# speedup vs baseline: 1.5428x; 1.5428x over previous
import jax
import jax.numpy as jnp
from jax import lax
from jax.experimental import pallas as pl
from jax.experimental.pallas import tpu as pltpu
from jax.experimental.pallas import tpu_sc as plsc

BATCH = 4
SEQ = 2048
D = 768
TOKENS = BATCH * SEQ
NUM_WORKERS = 32
S_PER_W = SEQ // NUM_WORKERS
SCH = 8
ROWS = BATCH * SCH
NUM_CHUNKS = S_PER_W // SCH
NBUF = 3
AHEAD = 2

_mesh = plsc.VectorSubcoreMesh(core_axis_name="c", subcore_axis_name="s")

_scratch = (
    [pltpu.VMEM((S_PER_W * BATCH,), jnp.int32)]
    + [pltpu.VMEM((S_PER_W, D), jnp.float32)]
    + [pltpu.VMEM((ROWS, D), jnp.float32) for _ in range(NBUF)]
    + [pltpu.SemaphoreType.DMA for _ in range(NBUF)]
    + [pltpu.SemaphoreType.DMA for _ in range(NBUF)]
    + [pltpu.SemaphoreType.DMA]
    + [pltpu.SemaphoreType.DMA]
)


@pl.kernel(
    out_type=jax.ShapeDtypeStruct((TOKENS, D), jnp.float32),
    mesh=_mesh,
    scratch_types=_scratch,
)
def _embed(ids_hbm, table_hbm, pos_hbm, out_hbm, idx_v, pbuf,
           b0, b1, b2, gs0, gs1, gs2, ws0, ws1, ws2, isem, psem):
    bufs = (b0, b1, b2)
    gsem = (gs0, gs1, gs2)
    wsem = (ws0, ws1, ws2)
    wid = lax.axis_index("s") * 2 + lax.axis_index("c")
    s_base = wid * S_PER_W

    id_copies = [
        pltpu.async_copy(
            ids_hbm.at[pl.ds(b * SEQ + s_base, S_PER_W)],
            idx_v.at[pl.ds(b * S_PER_W, S_PER_W)], isem)
        for b in range(BATCH)
    ]
    pos_copy = pltpu.async_copy(pos_hbm.at[pl.ds(s_base, S_PER_W)], pbuf, psem)
    for c in id_copies:
        c.wait()

    def start_gathers(j):
        slot = j % NBUF
        return [
            pltpu.async_copy(
                table_hbm.at[idx_v.at[pl.ds(b * S_PER_W + j * SCH, SCH)]],
                bufs[slot].at[pl.ds(b * SCH, SCH)], gsem[slot],
            )
            for b in range(BATCH)
        ]

    def start_writes(j):
        slot = j % NBUF
        return [
            pltpu.async_copy(
                bufs[slot].at[pl.ds(b * SCH, SCH)],
                out_hbm.at[pl.ds(b * SEQ + s_base + j * SCH, SCH)],
                wsem[slot],
            )
            for b in range(BATCH)
        ]

    gathers = [None] * NUM_CHUNKS
    writes = [None] * NUM_CHUNKS
    for j in range(min(AHEAD, NUM_CHUNKS)):
        gathers[j] = start_gathers(j)
    pos_copy.wait()

    for j in range(NUM_CHUNKS):
        slot = j % NBUF
        for g in gathers[j]:
            g.wait()
        buf = bufs[slot]

        def add_srow(r, _):
            prow = j * SCH + r
            for k in range(D // 16):
                sl = pl.ds(k * 16, 16)
                pvec = pbuf[prow, sl]
                for b in range(BATCH):
                    plsc.addupdate(buf.at[b * SCH + r, sl], pvec)
            return 0

        lax.fori_loop(0, SCH, add_srow, 0, unroll=False)
        writes[j] = start_writes(j)
        nxt = j + AHEAD
        if nxt < NUM_CHUNKS:
            prev = nxt - NBUF
            if prev >= 0:
                for w in writes[prev]:
                    w.wait()
            gathers[nxt] = start_gathers(nxt)

    for j in range(max(0, NUM_CHUNKS - NBUF), NUM_CHUNKS):
        for w in writes[j]:
            w.wait()


def kernel(token_ids, token_table, pos_table):
    out = _embed(token_ids.reshape(TOKENS), token_table, pos_table)
    return out.reshape(BATCH, SEQ, D)


# speedup vs baseline: 1.7013x; 1.1027x over previous
import jax
import jax.numpy as jnp
from jax import lax
from jax.experimental import pallas as pl
from jax.experimental.pallas import tpu as pltpu
from jax.experimental.pallas import tpu_sc as plsc

BATCH = 4
SEQ = 2048
D = 768
TOKENS = BATCH * SEQ
NUM_WORKERS = 32
S_PER_W = SEQ // NUM_WORKERS
SCH = 8
ROWS = BATCH * SCH
NUM_CHUNKS = S_PER_W // SCH
NBUF = 3
AHEAD = 2

_mesh = plsc.VectorSubcoreMesh(core_axis_name="c", subcore_axis_name="s")

_scratch = (
    [pltpu.VMEM((S_PER_W * BATCH,), jnp.int32)]
    + [pltpu.VMEM((S_PER_W, D), jnp.float32)]
    + [pltpu.VMEM((NBUF * ROWS, D), jnp.float32)]
    + [pltpu.SemaphoreType.DMA((NBUF,))]
    + [pltpu.SemaphoreType.DMA((NBUF,))]
    + [pltpu.SemaphoreType.DMA]
    + [pltpu.SemaphoreType.DMA]
)


@pl.kernel(
    out_type=jax.ShapeDtypeStruct((TOKENS, D), jnp.float32),
    mesh=_mesh,
    scratch_types=_scratch,
)
def _embed(ids_hbm, table_hbm, pos_hbm, out_hbm, idx_v, pbuf, ring,
           gsem, wsem, isem, psem):
    wid = lax.axis_index("s") * 2 + lax.axis_index("c")
    s_base = wid * S_PER_W

    id_copies = [
        pltpu.async_copy(
            ids_hbm.at[pl.ds(b * SEQ + s_base, S_PER_W)],
            idx_v.at[pl.ds(b * S_PER_W, S_PER_W)], isem)
        for b in range(BATCH)
    ]
    pos_copy = pltpu.async_copy(pos_hbm.at[pl.ds(s_base, S_PER_W)], pbuf, psem)
    for c in id_copies:
        c.wait()

    def gather_copies(j, slot):
        return [
            pltpu.make_async_copy(
                table_hbm.at[idx_v.at[pl.ds(b * S_PER_W + j * SCH, SCH)]],
                ring.at[pl.ds(slot * ROWS + b * SCH, SCH)], gsem.at[slot],
            )
            for b in range(BATCH)
        ]

    def write_copies(j, slot):
        return [
            pltpu.make_async_copy(
                ring.at[pl.ds(slot * ROWS + b * SCH, SCH)],
                out_hbm.at[pl.ds(b * SEQ + s_base + j * SCH, SCH)],
                wsem.at[slot],
            )
            for b in range(BATCH)
        ]

    for j in range(AHEAD):
        for c in gather_copies(j, j % NBUF):
            c.start()
    pos_copy.wait()

    def chunk_step(j, _):
        slot = lax.rem(j, NBUF)
        for c in gather_copies(j, slot):
            c.wait()

        def add_srow(r, _):
            prow = j * SCH + r
            for k in range(D // 16):
                sl = pl.ds(k * 16, 16)
                pvec = pbuf[prow, sl]
                for b in range(BATCH):
                    plsc.addupdate(ring.at[slot * ROWS + b * SCH + r, sl], pvec)
            return 0

        lax.fori_loop(0, SCH, add_srow, 0, unroll=False)
        for c in write_copies(j, slot):
            c.start()

        nxt = j + AHEAD
        nslot = lax.rem(nxt, NBUF)

        @pl.when(nxt < NUM_CHUNKS)
        def _():
            @pl.when(nxt >= NBUF)
            def _():
                for c in write_copies(nxt - NBUF, nslot):
                    c.wait()
            for c in gather_copies(nxt, nslot):
                c.start()

        return 0

    lax.fori_loop(0, NUM_CHUNKS, chunk_step, 0, unroll=False)

    for j in range(NUM_CHUNKS - NBUF, NUM_CHUNKS):
        for c in write_copies(j, j % NBUF):
            c.wait()


def kernel(token_ids, token_table, pos_table):
    out = _embed(token_ids.reshape(TOKENS), token_table, pos_table)
    return out.reshape(BATCH, SEQ, D)


# speedup vs baseline: 1.7019x; 1.0003x over previous
import jax
import jax.numpy as jnp
from jax import lax
from jax.experimental import pallas as pl
from jax.experimental.pallas import tpu as pltpu
from jax.experimental.pallas import tpu_sc as plsc

BATCH = 4
SEQ = 2048
D = 768
TOKENS = BATCH * SEQ
NUM_WORKERS = 32
S_PER_W = SEQ // NUM_WORKERS
SCH = 8
ROWS = BATCH * SCH
NUM_CHUNKS = S_PER_W // SCH
NBUF = 3
AHEAD = 2

_mesh = plsc.VectorSubcoreMesh(core_axis_name="c", subcore_axis_name="s")

_scratch = (
    [pltpu.VMEM((S_PER_W * BATCH,), jnp.int32)]
    + [pltpu.VMEM((S_PER_W, D), jnp.float32)]
    + [pltpu.VMEM((NBUF * ROWS, D), jnp.float32)]
    + [pltpu.SemaphoreType.DMA((NBUF,))]
    + [pltpu.SemaphoreType.DMA((NBUF,))]
    + [pltpu.SemaphoreType.DMA]
    + [pltpu.SemaphoreType.DMA]
)


@pl.kernel(
    out_type=jax.ShapeDtypeStruct((TOKENS, D), jnp.float32),
    mesh=_mesh,
    scratch_types=_scratch,
)
def _embed(ids_hbm, table_hbm, pos_hbm, out_hbm, idx_v, pbuf, ring,
           gsem, wsem, isem, psem):
    wid = lax.axis_index("s") * 2 + lax.axis_index("c")
    s_base = wid * S_PER_W

    id_copies = [
        pltpu.async_copy(
            ids_hbm.at[pl.ds(b * SEQ + s_base, S_PER_W)],
            idx_v.at[pl.ds(b * S_PER_W, S_PER_W)], isem)
        for b in range(BATCH)
    ]
    pos_copy = pltpu.async_copy(pos_hbm.at[pl.ds(s_base, S_PER_W)], pbuf, psem)
    for c in id_copies:
        c.wait()

    def gather_copies(j, slot):
        return [
            pltpu.make_async_copy(
                table_hbm.at[idx_v.at[pl.ds(b * S_PER_W + j * SCH, SCH)]],
                ring.at[pl.ds(slot * ROWS + b * SCH, SCH)], gsem.at[slot],
            )
            for b in range(BATCH)
        ]

    def write_copies(j, slot):
        return [
            pltpu.make_async_copy(
                ring.at[pl.ds(slot * ROWS + b * SCH, SCH)],
                out_hbm.at[pl.ds(b * SEQ + s_base + j * SCH, SCH)],
                wsem.at[slot],
            )
            for b in range(BATCH)
        ]

    for j in range(AHEAD):
        for c in gather_copies(j, j % NBUF):
            c.start()
    pos_copy.wait()

    def chunk_step(j, _):
        slot = lax.rem(j, NBUF)
        for c in gather_copies(j, slot):
            c.wait()

        def add_srow(r, _):
            prow = j * SCH + r
            for k in range(D // 16):
                sl = pl.ds(k * 16, 16)
                pvec = pbuf[prow, sl]
                for b in range(BATCH):
                    plsc.addupdate(ring.at[slot * ROWS + b * SCH + r, sl], pvec)
            return 0

        lax.fori_loop(0, SCH, add_srow, 0, unroll=False)

        nxt = j + AHEAD
        nslot = lax.rem(nxt, NBUF)

        @pl.when(nxt < NUM_CHUNKS)
        def _():
            @pl.when(nxt >= NBUF)
            def _():
                for c in write_copies(nxt - NBUF, nslot):
                    c.wait()
            for c in gather_copies(nxt, nslot):
                c.start()

        for c in write_copies(j, slot):
            c.start()
        return 0

    lax.fori_loop(0, NUM_CHUNKS, chunk_step, 0, unroll=False)

    for j in range(NUM_CHUNKS - NBUF, NUM_CHUNKS):
        for c in write_copies(j, j % NBUF):
            c.wait()


def kernel(token_ids, token_table, pos_table):
    out = _embed(token_ids.reshape(TOKENS), token_table, pos_table)
    return out.reshape(BATCH, SEQ, D)
